# baseline (speedup 1.0000x reference)
import jax
import jax.numpy as jnp
from jax import lax
from jax.experimental import pallas as pl
from jax.experimental.pallas import tpu as pltpu
from jax.experimental.pallas import tpu_sc as plsc

NUM_K = 8192
DIM = 256
ROWS = 256
NROWS = 8192
NWORK = 32
HALF_STEPS = (NROWS // ROWS) // 2


def _vq_body(h_ref, cb_ref, q_ref, idx_ref, cbn_ref):
    @pl.when(pl.program_id(0) == 0)
    def _():
        x = cb_ref[...]
        cn = jnp.sqrt(jnp.sum(x * x, axis=-1, keepdims=True))
        cbn_ref[...] = x / jnp.maximum(cn, 1e-6)

    hb = h_ref[...]
    n = jnp.sqrt(jnp.sum(hb * hb, axis=-1, keepdims=True))
    hn = hb / jnp.maximum(n, 1e-6)
    cbn = cbn_ref[...]
    sims = lax.dot_general(hn, cbn, (((1,), (1,)), ((), ())),
                           preferred_element_type=jnp.float32)
    d = 2.0 - 2.0 * sims
    m = jnp.min(d, axis=1, keepdims=True)
    iota_f = lax.broadcasted_iota(jnp.int32, d.shape, 1).astype(jnp.float32)
    cand = jnp.where(d == m, iota_f, float(NUM_K))
    idx_f = jnp.min(cand, axis=1)
    q_ref[...] = jnp.where(cand == idx_f[:, None], 1.0, 0.0)
    idx_ref[0, 0, :] = idx_f.astype(jnp.int32)


def _vq_body_alias(h_ref, cb_ref, qprev_ref, q_ref, idx_ref, cbn_ref):
    del qprev_ref
    _vq_body(h_ref, cb_ref, q_ref, idx_ref, cbn_ref)


def _gather_body(cbn_hbm, idx_hbm, out_hbm, idx_v, rows_v, sem):
    rpw = idx_hbm.shape[0] // NWORK
    wid = lax.axis_index("s") * 2 + lax.axis_index("c")
    base = wid * rpw
    pltpu.sync_copy(idx_hbm.at[pl.ds(base, rpw)], idx_v)
    pltpu.async_copy(cbn_hbm.at[idx_v], rows_v, sem).wait()
    pltpu.sync_copy(rows_v, out_hbm.at[pl.ds(base, rpw)])


def _vq_part(h_flat, codebook, offset, q_prev):
    if q_prev is None:
        body = _vq_body
        in_specs = [
            pl.BlockSpec((ROWS, DIM), lambda i, o=offset: (i + o, 0)),
            pl.BlockSpec((NUM_K, DIM), lambda i: (0, 0)),
        ]
        operands = (h_flat, codebook)
        aliases = {}
    else:
        body = _vq_body_alias
        in_specs = [
            pl.BlockSpec((ROWS, DIM), lambda i, o=offset: (i + o, 0)),
            pl.BlockSpec((NUM_K, DIM), lambda i: (0, 0)),
            pl.BlockSpec(memory_space=pl.ANY),
        ]
        operands = (h_flat, codebook, q_prev)
        aliases = {2: 0}
    return pl.pallas_call(
        body,
        grid=(HALF_STEPS,),
        in_specs=in_specs,
        out_specs=[
            pl.BlockSpec((ROWS, NUM_K), lambda i, o=offset: (i + o, 0)),
            pl.BlockSpec((1, 1, ROWS), lambda i: (i, 0, 0)),
            pl.BlockSpec((NUM_K, DIM), lambda i: (0, 0)),
        ],
        out_shape=[
            jax.ShapeDtypeStruct((NROWS, NUM_K), jnp.float32),
            jax.ShapeDtypeStruct((HALF_STEPS, 1, ROWS), jnp.int32),
            jax.ShapeDtypeStruct((NUM_K, DIM), jnp.float32),
        ],
        input_output_aliases=aliases,
    )(*operands)


def _sc_gather(cbn, idx_flat):
    nr = idx_flat.shape[0]
    mesh = plsc.VectorSubcoreMesh(core_axis_name="c", subcore_axis_name="s")
    f = pl.kernel(
        _gather_body,
        mesh=mesh,
        out_type=jax.ShapeDtypeStruct((nr, DIM), jnp.float32),
        scratch_types=[
            pltpu.VMEM((nr // NWORK,), jnp.int32),
            pltpu.VMEM((nr // NWORK, DIM), jnp.float32),
            pltpu.SemaphoreType.DMA,
        ],
    )
    return f(cbn, idx_flat)


def kernel(h, codebook):
    B, S, D = h.shape
    half = NROWS // 2
    h_flat = h.reshape(-1, D)
    _q0, idx0, cbn = _vq_part(h_flat, codebook, 0, None)
    q_full, idx1, _ = _vq_part(h_flat, codebook, HALF_STEPS, _q0)
    c0 = _sc_gather(cbn, idx0.reshape(half))
    c1 = _sc_gather(cbn, idx1.reshape(half))
    q = q_full.reshape(B, S, NUM_K)
    c = jnp.concatenate([c0, c1], axis=0).reshape(B, S, D)
    indices = jnp.concatenate(
        [idx0.reshape(half), idx1.reshape(half)]).reshape(B, S)
    return (q, c, c, c, indices)


# speedup vs baseline: 1.0985x; 1.0985x over previous
import jax
import jax.numpy as jnp
from jax import lax
from jax.experimental import pallas as pl
from jax.experimental.pallas import tpu as pltpu
from jax.experimental.pallas import tpu_sc as plsc

NUM_K = 8192
DIM = 256
ROWS = 512
NROWS = 8192
NWORK = 32
RPW = NROWS // NWORK


def _vq_body(h_ref, cb_hbm, q_ref, idx_ref, cbn_ref, sem):
    @pl.when(pl.program_id(0) == 0)
    def _():
        cp = pltpu.make_async_copy(cb_hbm, cbn_ref, sem)
        cp.start()
        cp.wait()
        x = cbn_ref[...]
        cn = jnp.sqrt(jnp.sum(x * x, axis=-1, keepdims=True))
        cbn_ref[...] = x / jnp.maximum(cn, 1e-6)

    hb = h_ref[...]
    n = jnp.sqrt(jnp.sum(hb * hb, axis=-1, keepdims=True))
    hn = hb / jnp.maximum(n, 1e-6)
    cbn = cbn_ref[...]
    sims = lax.dot_general(hn, cbn, (((1,), (1,)), ((), ())),
                           preferred_element_type=jnp.float32)
    d = 2.0 - 2.0 * sims
    m = jnp.min(d, axis=1, keepdims=True)
    iota_f = lax.broadcasted_iota(jnp.int32, d.shape, 1).astype(jnp.float32)
    cand = jnp.where(d == m, iota_f, float(NUM_K))
    idx_f = jnp.min(cand, axis=1)
    q_ref[...] = jnp.where(cand == idx_f[:, None], 1.0, 0.0)
    idx_ref[0, 0, :] = idx_f.astype(jnp.int32)


def _gather_body(cbn_hbm, idx_hbm, out_hbm, idx_v, rows_v, sem):
    wid = lax.axis_index("s") * 2 + lax.axis_index("c")
    base = wid * RPW
    pltpu.sync_copy(idx_hbm.at[pl.ds(base, RPW)], idx_v)
    pltpu.async_copy(cbn_hbm.at[idx_v], rows_v, sem).wait()
    pltpu.sync_copy(rows_v, out_hbm.at[pl.ds(base, RPW)])


def _vq(h_flat, codebook):
    return pl.pallas_call(
        _vq_body,
        grid=(NROWS // ROWS,),
        in_specs=[
            pl.BlockSpec((ROWS, DIM), lambda i: (i, 0)),
            pl.BlockSpec(memory_space=pl.ANY),
        ],
        scratch_shapes=[pltpu.SemaphoreType.DMA],
        out_specs=[
            pl.BlockSpec((ROWS, NUM_K), lambda i: (i, 0)),
            pl.BlockSpec((1, 1, ROWS), lambda i: (i, 0, 0)),
            pl.BlockSpec((NUM_K, DIM), lambda i: (0, 0)),
        ],
        out_shape=[
            jax.ShapeDtypeStruct((NROWS, NUM_K), jnp.float32),
            jax.ShapeDtypeStruct((NROWS // ROWS, 1, ROWS), jnp.int32),
            jax.ShapeDtypeStruct((NUM_K, DIM), jnp.float32),
        ],
    )(h_flat, codebook)


def _sc_gather(cbn, idx_flat):
    mesh = plsc.VectorSubcoreMesh(core_axis_name="c", subcore_axis_name="s")
    f = pl.kernel(
        _gather_body,
        mesh=mesh,
        out_type=jax.ShapeDtypeStruct((NROWS, DIM), jnp.float32),
        scratch_types=[
            pltpu.VMEM((RPW,), jnp.int32),
            pltpu.VMEM((RPW, DIM), jnp.float32),
            pltpu.SemaphoreType.DMA,
        ],
    )
    return f(cbn, idx_flat)


def kernel(h, codebook):
    B, S, D = h.shape
    h_flat = h.reshape(-1, D)
    q_flat, idx3, cbn = _vq(h_flat, codebook)
    idx_flat = idx3.reshape(NROWS)
    c_flat = _sc_gather(cbn, idx_flat)
    q = q_flat.reshape(B, S, NUM_K)
    c = c_flat.reshape(B, S, D)
    indices = idx3.reshape(B, S)
    return (q, c, c, c, indices)


# speedup vs baseline: 1.1046x; 1.0055x over previous
import jax
import jax.numpy as jnp
from jax import lax
from jax.experimental import pallas as pl
from jax.experimental.pallas import tpu as pltpu
from jax.experimental.pallas import tpu_sc as plsc

NUM_K = 8192
DIM = 256
ROWS = 256
NROWS = 8192
NWORK = 32
RPW = NROWS // NWORK


def _vq_body(h_ref, cb_hbm, q_ref, idx_ref, cbn_ref, sem):
    @pl.when(pl.program_id(0) == 0)
    def _():
        cp = pltpu.make_async_copy(cb_hbm, cbn_ref, sem)
        cp.start()
        cp.wait()
        x = cbn_ref[...]
        cn = jnp.sqrt(jnp.sum(x * x, axis=-1, keepdims=True))
        cbn_ref[...] = x / jnp.maximum(cn, 1e-6)

    hb = h_ref[...]
    n = jnp.sqrt(jnp.sum(hb * hb, axis=-1, keepdims=True))
    hn = hb / jnp.maximum(n, 1e-6)
    cbn = cbn_ref[...]
    sims = lax.dot_general(hn, cbn, (((1,), (1,)), ((), ())),
                           preferred_element_type=jnp.float32)
    d = 2.0 - 2.0 * sims
    m = jnp.min(d, axis=1, keepdims=True)
    iota_f = lax.broadcasted_iota(jnp.int32, d.shape, 1).astype(jnp.float32)
    cand = jnp.where(d == m, iota_f, float(NUM_K))
    idx_f = jnp.min(cand, axis=1)
    q_ref[...] = jnp.where(cand == idx_f[:, None], 1.0, 0.0)
    idx_ref[0, 0, :] = idx_f.astype(jnp.int32)


def _gather_body(cbn_hbm, idx_hbm, out_hbm, idx_v, rows_v, sem):
    wid = lax.axis_index("s") * 2 + lax.axis_index("c")
    base = wid * RPW
    pltpu.sync_copy(idx_hbm.at[pl.ds(base, RPW)], idx_v)
    pltpu.async_copy(cbn_hbm.at[idx_v], rows_v, sem).wait()
    pltpu.sync_copy(rows_v, out_hbm.at[pl.ds(base, RPW)])


def _vq(h_flat, codebook):
    return pl.pallas_call(
        _vq_body,
        grid=(NROWS // ROWS,),
        in_specs=[
            pl.BlockSpec((ROWS, DIM), lambda i: (i, 0)),
            pl.BlockSpec(memory_space=pl.ANY),
        ],
        scratch_shapes=[pltpu.SemaphoreType.DMA],
        out_specs=[
            pl.BlockSpec((ROWS, NUM_K), lambda i: (i, 0)),
            pl.BlockSpec((1, 1, ROWS), lambda i: (i, 0, 0)),
            pl.BlockSpec((NUM_K, DIM), lambda i: (0, 0)),
        ],
        out_shape=[
            jax.ShapeDtypeStruct((NROWS, NUM_K), jnp.float32),
            jax.ShapeDtypeStruct((NROWS // ROWS, 1, ROWS), jnp.int32),
            jax.ShapeDtypeStruct((NUM_K, DIM), jnp.float32),
        ],
    )(h_flat, codebook)


def _sc_gather(cbn, idx_flat):
    mesh = plsc.VectorSubcoreMesh(core_axis_name="c", subcore_axis_name="s")
    f = pl.kernel(
        _gather_body,
        mesh=mesh,
        out_type=jax.ShapeDtypeStruct((NROWS, DIM), jnp.float32),
        scratch_types=[
            pltpu.VMEM((RPW,), jnp.int32),
            pltpu.VMEM((RPW, DIM), jnp.float32),
            pltpu.SemaphoreType.DMA,
        ],
    )
    return f(cbn, idx_flat)


def kernel(h, codebook):
    B, S, D = h.shape
    h_flat = h.reshape(-1, D)
    q_flat, idx3, cbn = _vq(h_flat, codebook)
    idx_flat = idx3.reshape(NROWS)
    c_flat = _sc_gather(cbn, idx_flat)
    q = q_flat.reshape(B, S, NUM_K)
    c = c_flat.reshape(B, S, D)
    indices = idx3.reshape(B, S)
    return (q, c, c, c, indices)
